# Initial kernel scaffold; baseline (speedup 1.0000x reference)
import jax
import jax.numpy as jnp
from jax.experimental import pallas as pl


def kernel(speaker_emb, cond_prompt_speech_tokens, emotion_adv, text_tokens, speech_tokens, text_emb_table, speech_emb_table, text_pos_table, speech_pos_table, spkr_W, spkr_b, emo_W, emo_b):
    raise NotImplementedError("write your pallas kernel here")


# speedup vs baseline: 1.6365x; 1.6365x over previous
import functools

import jax
import jax.numpy as jnp
from jax import lax
from jax.experimental import pallas as pl
from jax.experimental.pallas import tpu as pltpu
from jax.experimental.pallas import tpu_sc as plsc

B = 16
C = 1024
T_PROMPT = 150
T_TEXT = 1024
T_SPEECH = 2048
LEN_COND = 152
T_OUT = LEN_COND + T_TEXT + T_SPEECH

NC = 2
NS = 16
NW = NC * NS

W_P = 75
W_PG = 80
W_T = 64
W_S = 64


def _cond_rows_tc(speaker_emb, emotion_adv, spkr_W, spkr_b, emo_W, emo_b):
    def body(spk, emo, w_ref, b_ref, ew_ref, eb_ref, out):
        proj = jnp.dot(spk[...], w_ref[...], preferred_element_type=jnp.float32)
        out[0] = proj + b_ref[...]
        out[1] = emo[...] * ew_ref[...] + eb_ref[...]

    return pl.pallas_call(
        body,
        out_shape=jax.ShapeDtypeStruct((2, B, C), jnp.float32),
    )(speaker_emb, emotion_adv.reshape(B, 1), spkr_W,
      spkr_b.reshape(1, C), emo_W, emo_b.reshape(1, C))


def _sc_embed(cond2, p_idx, t_idx, s_idx, t_tab, s_tab, t_pos, s_pos):
    mesh = plsc.VectorSubcoreMesh(
        core_axis_name="c", subcore_axis_name="s", num_cores=NC, num_subcores=NS)

    @functools.partial(
        pl.kernel,
        out_type=jax.ShapeDtypeStruct((B * T_OUT, 8, 128), jnp.float32),
        mesh=mesh,
        scratch_types=[
            pltpu.VMEM((W_PG,), jnp.int32),
            pltpu.VMEM((W_T,), jnp.int32),
            pltpu.VMEM((W_PG, 8, 128), jnp.float32),
            pltpu.SemaphoreType.DMA,
        ],
    )
    def body(cond2_h, p_idx_h, t_idx_h, s_idx_h, t_tab_h, s_tab_h, t_pos_h,
             s_pos_h, out_h, idx_p, idx_c, buf, sem):
        w = lax.axis_index("s") * NC + lax.axis_index("c")
        b = w // 2
        h = w % 2
        out_b = b * T_OUT

        pltpu.sync_copy(cond2_h.at[pl.ds(h * B + b, 1)], buf.at[pl.ds(0, 1)])
        pltpu.sync_copy(buf.at[pl.ds(0, 1)],
                        out_h.at[pl.ds(out_b + h * (LEN_COND - 1), 1)])

        pltpu.sync_copy(p_idx_h.at[pl.ds(w * W_PG, W_PG)], idx_p)
        pltpu.sync_copy(s_pos_h.at[pl.ds(h * W_P, W_PG)], buf)
        pltpu.async_copy(s_tab_h.at[idx_p], buf, sem, add=True).wait()
        pltpu.sync_copy(buf.at[pl.ds(0, W_P)],
                        out_h.at[pl.ds(out_b + 1 + h * W_P, W_P)])

        for i in range(T_TEXT * B // W_T // NW):
            task = w + NW * i
            tb = task // (T_TEXT // W_T)
            tc = task % (T_TEXT // W_T)
            pltpu.sync_copy(t_idx_h.at[pl.ds(task * W_T, W_T)], idx_c)
            pltpu.sync_copy(t_pos_h.at[pl.ds(tc * W_T, W_T)], buf.at[pl.ds(0, W_T)])
            pltpu.async_copy(t_tab_h.at[idx_c], buf.at[pl.ds(0, W_T)], sem,
                             add=True).wait()
            pltpu.sync_copy(buf.at[pl.ds(0, W_T)],
                            out_h.at[pl.ds(tb * T_OUT + LEN_COND + tc * W_T, W_T)])

        for i in range(T_SPEECH * B // W_S // NW):
            task = w + NW * i
            sb = task // (T_SPEECH // W_S)
            sc = task % (T_SPEECH // W_S)
            pltpu.sync_copy(s_idx_h.at[pl.ds(task * W_S, W_S)], idx_c)
            pltpu.sync_copy(s_pos_h.at[pl.ds(sc * W_S, W_S)], buf.at[pl.ds(0, W_S)])
            pltpu.async_copy(s_tab_h.at[idx_c], buf.at[pl.ds(0, W_S)], sem,
                             add=True).wait()
            pltpu.sync_copy(
                buf.at[pl.ds(0, W_S)],
                out_h.at[pl.ds(sb * T_OUT + LEN_COND + T_TEXT + sc * W_S, W_S)])

    return body(cond2, p_idx, t_idx, s_idx, t_tab, s_tab, t_pos, s_pos)


def kernel(speaker_emb, cond_prompt_speech_tokens, emotion_adv, text_tokens,
           speech_tokens, text_emb_table, speech_emb_table, text_pos_table,
           speech_pos_table, spkr_W, spkr_b, emo_W, emo_b):
    cond2 = _cond_rows_tc(speaker_emb, emotion_adv, spkr_W, spkr_b, emo_W, emo_b)

    p_idx = jnp.pad(
        cond_prompt_speech_tokens.astype(jnp.int32).reshape(B * 2, W_P),
        ((0, 0), (0, W_PG - W_P))).reshape(-1)
    t_idx = text_tokens.astype(jnp.int32).reshape(-1)
    s_idx = speech_tokens.astype(jnp.int32).reshape(-1)

    out = _sc_embed(
        cond2.reshape(2 * B, 8, 128), p_idx, t_idx, s_idx,
        text_emb_table.reshape(-1, 8, 128), speech_emb_table.reshape(-1, 8, 128),
        text_pos_table.reshape(-1, 8, 128), speech_pos_table.reshape(-1, 8, 128))
    return (out.reshape(B, T_OUT, C), LEN_COND)
